# baseline (speedup 1.0000x reference)
import jax
import jax.numpy as jnp
from jax import lax
from jax.experimental import pallas as pl
from jax.experimental.pallas import tpu as pltpu
from jax.experimental.pallas import tpu_sc as plsc

EMB_DIM = 64
NC = 2
NS = 16
NW = NC * NS


HALF = 512


def _gather_body(table_hbm, idx_hbm, out_hbm, idx_v, rows_v, sem):
    wid = lax.axis_index("s") * NC + lax.axis_index("c")
    b_per_w = idx_v.shape[0]
    base = wid * b_per_w
    pltpu.sync_copy(idx_hbm.at[pl.ds(base, b_per_w)], idx_v)

    for h in range(b_per_w // HALF):
        def body(g, carry, h=h):
            o = pl.multiple_of(h * HALF + g * 16, 16)
            d = pl.multiple_of(g * 16, 16)
            v = idx_v[pl.ds(o, 16)] * EMB_DIM
            for j in range(16):
                off = pl.multiple_of(v[j], EMB_DIM)
                pltpu.async_copy(
                    table_hbm.at[pl.ds(off, EMB_DIM)],
                    rows_v.at[pl.ds((d + j) * EMB_DIM, EMB_DIM)],
                    sem,
                )
            return carry

        lax.fori_loop(0, HALF // 16, body, 0)
        pltpu.make_async_copy(
            table_hbm.at[pl.ds(0, HALF * EMB_DIM)], rows_v, sem
        ).wait()
        pltpu.sync_copy(
            rows_v, out_hbm.at[pl.ds((base + h * HALF) * EMB_DIM, HALF * EMB_DIM)]
        )


def _sc_gather(table, idx):
    n = idx.shape[0]
    b_per_w = n // NW
    mesh = plsc.VectorSubcoreMesh(core_axis_name="c", subcore_axis_name="s")
    k = pl.kernel(
        _gather_body,
        out_type=jax.ShapeDtypeStruct((n * EMB_DIM,), jnp.float32),
        mesh=mesh,
        scratch_types=[
            pltpu.VMEM((b_per_w,), jnp.int32),
            pltpu.VMEM((HALF * EMB_DIM,), jnp.float32),
            pltpu.SemaphoreType.DMA,
        ],
    )
    return k(table.reshape(-1), idx)


def _mlp_body(eu_ref, ev_ref, w1u_ref, w1v_ref, b1_ref, w2_ref, b2_ref, out_ref):
    h = (
        jnp.dot(eu_ref[...], w1u_ref[...], preferred_element_type=jnp.float32)
        + jnp.dot(ev_ref[...], w1v_ref[...], preferred_element_type=jnp.float32)
        + b1_ref[...]
    )
    h = jnp.maximum(h, 0.0)
    o = jnp.dot(h, w2_ref[...], preferred_element_type=jnp.float32) + b2_ref[0, 0]
    out_ref[...] = jax.nn.sigmoid(o)


def _mlp(g, w1u, w1v, b1r, w2t, b2r, batch, blk):
    nb = batch // blk
    return pl.pallas_call(
        _mlp_body,
        grid=(nb,),
        in_specs=[
            pl.BlockSpec((blk, EMB_DIM), lambda i: (i, 0)),
            pl.BlockSpec((blk, EMB_DIM), lambda i, nb=nb: (i + nb, 0)),
            pl.BlockSpec((EMB_DIM, EMB_DIM), lambda i: (0, 0)),
            pl.BlockSpec((EMB_DIM, EMB_DIM), lambda i: (0, 0)),
            pl.BlockSpec((1, EMB_DIM), lambda i: (0, 0)),
            pl.BlockSpec((EMB_DIM, 1), lambda i: (0, 0)),
            pl.BlockSpec((1, 1), lambda i: (0, 0)),
        ],
        out_specs=pl.BlockSpec((blk, 1), lambda i: (i, 0)),
        out_shape=jax.ShapeDtypeStruct((batch, 1), jnp.float32),
    )(g, g, w1u, w1v, b1r, w2t, b2r)


def kernel(u_ids, v_ids, emb, W1, b1, W2, b2):
    batch = u_ids.shape[0]
    idx = jnp.concatenate([u_ids.astype(jnp.int32), v_ids.astype(jnp.int32)])
    g = _sc_gather(emb, idx).reshape(2 * batch, EMB_DIM)
    w1u = W1[:, :EMB_DIM].T
    w1v = W1[:, EMB_DIM:].T
    out = _mlp(
        g, w1u, w1v,
        b1.reshape(1, EMB_DIM), W2.T, b2.reshape(1, 1),
        batch, 1024,
    )
    return out[:, 0]


# speedup vs baseline: 1.8583x; 1.8583x over previous
import jax
import jax.numpy as jnp
from jax import lax
from jax.experimental import pallas as pl
from jax.experimental.pallas import tpu as pltpu
from jax.experimental.pallas import tpu_sc as plsc

EMB_DIM = 64
NC = 2
NS = 16
NW = NC * NS

BM = 8192
HALF = 512

def _rne_bf16(u):
    return ((u + 0x7FFF + ((u >> 16) & 1)) >> 16) << 16


def _proj_body(xT_ref, w_ref, out_ref):
    x = xT_ref[...].astype(jnp.bfloat16)
    w = w_ref[...].astype(jnp.bfloat16)
    y = lax.dot_general(
        x, w, (((0,), (0,)), ((), ())), preferred_element_type=jnp.float32
    )
    a = _rne_bf16(lax.bitcast_convert_type(y[:, :EMB_DIM], jnp.uint32))
    b = _rne_bf16(lax.bitcast_convert_type(y[:, EMB_DIM:], jnp.uint32))
    out_ref[...] = lax.bitcast_convert_type(a | (b >> 16), jnp.float32)


def _project(embT, wcat):
    n = embT.shape[1]
    return pl.pallas_call(
        _proj_body,
        grid=(pl.cdiv(n, BM),),
        in_specs=[
            pl.BlockSpec((EMB_DIM, BM), lambda i: (0, i)),
            pl.BlockSpec((EMB_DIM, 2 * EMB_DIM), lambda i: (0, 0)),
        ],
        out_specs=pl.BlockSpec((BM, EMB_DIM), lambda i: (i, 0)),
        out_shape=jax.ShapeDtypeStruct((n, EMB_DIM), jnp.float32),
    )(embT, wcat)


def _gather_body(table_hbm, idx_hbm, out_hbm, idx_v, rows_v, sem):
    wid = lax.axis_index("s") * NC + lax.axis_index("c")
    b_per_w = idx_v.shape[0]
    base = wid * b_per_w
    pltpu.sync_copy(idx_hbm.at[pl.ds(base, b_per_w)], idx_v)

    for h in range(b_per_w // HALF):
        def body(g, carry, h=h):
            o = pl.multiple_of(h * HALF + g * 16, 16)
            d = pl.multiple_of(g * 16, 16)
            v = idx_v[pl.ds(o, 16)]
            for j in range(16):
                row = v[j]
                pltpu.async_copy(
                    table_hbm.at[pl.ds(row, 1)], rows_v.at[pl.ds(d + j, 1)], sem
                )
            return carry

        lax.fori_loop(0, HALF // 16, body, 0)
        pltpu.make_async_copy(table_hbm.at[pl.ds(0, HALF)], rows_v, sem).wait()
        pltpu.sync_copy(rows_v, out_hbm.at[pl.ds(base + h * HALF, HALF)])


def _sc_gather(table, idx):
    n = idx.shape[0]
    b_per_w = n // NW
    mesh = plsc.VectorSubcoreMesh(core_axis_name="c", subcore_axis_name="s")
    k = pl.kernel(
        _gather_body,
        out_type=jax.ShapeDtypeStruct((n, EMB_DIM), jnp.float32),
        mesh=mesh,
        scratch_types=[
            pltpu.VMEM((b_per_w,), jnp.int32),
            pltpu.VMEM((HALF, EMB_DIM), jnp.float32),
            pltpu.SemaphoreType.DMA,
        ],
    )
    return k(table, idx)


def _mlp_body(eu_ref, ev_ref, b1_ref, w2_ref, b2_ref, out_ref):
    au = lax.bitcast_convert_type(eu_ref[...], jnp.uint32)
    av = lax.bitcast_convert_type(ev_ref[...], jnp.uint32)
    p = lax.bitcast_convert_type((au >> 16) << 16, jnp.float32)
    q = lax.bitcast_convert_type(av << 16, jnp.float32)
    h = jnp.maximum(p + q + b1_ref[...], 0.0)
    o = jnp.dot(h, w2_ref[...], preferred_element_type=jnp.float32) + b2_ref[0, 0]
    out_ref[...] = jax.nn.sigmoid(o)


def _mlp(g, b1r, w2t, b2r, batch, blk):
    nb = batch // blk
    return pl.pallas_call(
        _mlp_body,
        grid=(nb,),
        in_specs=[
            pl.BlockSpec((blk, EMB_DIM), lambda i: (i, 0)),
            pl.BlockSpec((blk, EMB_DIM), lambda i, nb=nb: (i + nb, 0)),
            pl.BlockSpec((1, EMB_DIM), lambda i: (0, 0)),
            pl.BlockSpec((EMB_DIM, 1), lambda i: (0, 0)),
            pl.BlockSpec((1, 1), lambda i: (0, 0)),
        ],
        out_specs=pl.BlockSpec((blk, 1), lambda i: (i, 0)),
        out_shape=jax.ShapeDtypeStruct((batch, 1), jnp.float32),
    )(g, g, b1r, w2t, b2r)


def kernel(u_ids, v_ids, emb, W1, b1, W2, b2):
    batch = u_ids.shape[0]
    idx = jnp.concatenate([u_ids.astype(jnp.int32), v_ids.astype(jnp.int32)])
    wcat = jnp.concatenate([W1[:, :EMB_DIM].T, W1[:, EMB_DIM:].T], axis=1)
    r32 = _project(emb.T, wcat)
    g = _sc_gather(r32, idx)
    out = _mlp(g, b1.reshape(1, EMB_DIM), W2.T, b2.reshape(1, 1), batch, 1024)
    return out[:, 0]


# speedup vs baseline: 2.0840x; 1.1214x over previous
import jax
import jax.numpy as jnp
from jax import lax
from jax.experimental import pallas as pl
from jax.experimental.pallas import tpu as pltpu
from jax.experimental.pallas import tpu_sc as plsc

EMB_DIM = 64
NC = 2
NS = 16
NW = NC * NS

BM = 8192
HALF = 512

def _transpose_body(xT_ref, out_ref):
    out_ref[...] = xT_ref[...].T


def _repack(embT):
    n = embT.shape[1]
    return pl.pallas_call(
        _transpose_body,
        grid=(pl.cdiv(n, BM),),
        in_specs=[pl.BlockSpec((EMB_DIM, BM), lambda i: (0, i))],
        out_specs=pl.BlockSpec((BM, EMB_DIM), lambda i: (i, 0)),
        out_shape=jax.ShapeDtypeStruct((n, EMB_DIM), jnp.float32),
    )(embT)


def _gather_body(table_hbm, idx_hbm, out_hbm, idx_v, rows_v, sem):
    wid = lax.axis_index("s") * NC + lax.axis_index("c")
    b_per_w = idx_v.shape[0]
    base = wid * b_per_w
    pltpu.sync_copy(idx_hbm.at[pl.ds(base, b_per_w)], idx_v)

    for h in range(b_per_w // HALF):
        def body(g, carry, h=h):
            o = pl.multiple_of(h * HALF + g * 16, 16)
            d = pl.multiple_of(g * 16, 16)
            v = idx_v[pl.ds(o, 16)]
            for j in range(16):
                row = v[j]
                pltpu.async_copy(
                    table_hbm.at[pl.ds(row, 1)], rows_v.at[pl.ds(d + j, 1)], sem
                )
            return carry

        lax.fori_loop(0, HALF // 16, body, 0)
        pltpu.make_async_copy(table_hbm.at[pl.ds(0, HALF)], rows_v, sem).wait()
        pltpu.sync_copy(rows_v, out_hbm.at[pl.ds(base + h * HALF, HALF)])


def _sc_gather(table, idx):
    n = idx.shape[0]
    b_per_w = n // NW
    mesh = plsc.VectorSubcoreMesh(core_axis_name="c", subcore_axis_name="s")
    k = pl.kernel(
        _gather_body,
        out_type=jax.ShapeDtypeStruct((n, EMB_DIM), jnp.float32),
        mesh=mesh,
        scratch_types=[
            pltpu.VMEM((b_per_w,), jnp.int32),
            pltpu.VMEM((HALF, EMB_DIM), jnp.float32),
            pltpu.SemaphoreType.DMA,
        ],
    )
    return k(table, idx)


def _mlp_body(eu_ref, ev_ref, w1u_ref, w1v_ref, b1_ref, w2_ref, b2_ref, out_ref):
    h = (
        jnp.dot(eu_ref[...], w1u_ref[...], preferred_element_type=jnp.float32)
        + jnp.dot(ev_ref[...], w1v_ref[...], preferred_element_type=jnp.float32)
        + b1_ref[...]
    )
    h = jnp.maximum(h, 0.0)
    o = jnp.dot(h, w2_ref[...], preferred_element_type=jnp.float32) + b2_ref[0, 0]
    out_ref[...] = jax.nn.sigmoid(o)


def _mlp(g, w1u, w1v, b1r, w2t, b2r, batch, blk):
    nb = batch // blk
    return pl.pallas_call(
        _mlp_body,
        grid=(nb,),
        in_specs=[
            pl.BlockSpec((blk, EMB_DIM), lambda i: (i, 0)),
            pl.BlockSpec((blk, EMB_DIM), lambda i, nb=nb: (i + nb, 0)),
            pl.BlockSpec((EMB_DIM, EMB_DIM), lambda i: (0, 0)),
            pl.BlockSpec((EMB_DIM, EMB_DIM), lambda i: (0, 0)),
            pl.BlockSpec((1, EMB_DIM), lambda i: (0, 0)),
            pl.BlockSpec((EMB_DIM, 1), lambda i: (0, 0)),
            pl.BlockSpec((1, 1), lambda i: (0, 0)),
        ],
        out_specs=pl.BlockSpec((blk, 1), lambda i: (i, 0)),
        out_shape=jax.ShapeDtypeStruct((batch, 1), jnp.float32),
    )(g, g, w1u, w1v, b1r, w2t, b2r)


def kernel(u_ids, v_ids, emb, W1, b1, W2, b2):
    batch = u_ids.shape[0]
    idx = jnp.concatenate([u_ids.astype(jnp.int32), v_ids.astype(jnp.int32)])
    table = _repack(emb.T)
    g = _sc_gather(table, idx)
    out = _mlp(
        g, W1[:, :EMB_DIM].T, W1[:, EMB_DIM:].T,
        b1.reshape(1, EMB_DIM), W2.T, b2.reshape(1, 1), batch, 1024,
    )
    return out[:, 0]
